# baseline (speedup 1.0000x reference)
import jax
import jax.numpy as jnp
from jax.experimental import pallas as pl
from jax.experimental.pallas import tpu as pltpu

HIDDEN = 2048
NUM_EXPERTS = 64
LOAD_BALANCING_LAMBDA = 0.01
TOKENS = 4 * 4096
BLOCK_T = 2048
N_STEPS = TOKENS // BLOCK_T


def _router_kernel(x_ref, w_ref, logits_ref, mask_ref, psum_ref, usum_ref):
    x = x_ref[...]
    w = w_ref[...]
    logits = jax.lax.dot_general(
        x, w, (((1,), (1,)), ((), ())), preferred_element_type=jnp.float32)
    logits_ref[...] = logits

    m = jnp.max(logits, axis=-1, keepdims=True)
    e = jnp.exp(logits - m)
    s = jnp.sum(e, axis=-1, keepdims=True)
    probs = e / s

    iota = jax.lax.broadcasted_iota(jnp.int32, logits.shape, 1)
    eq = logits == m
    idx = jnp.min(jnp.where(eq, iota, NUM_EXPERTS), axis=-1, keepdims=True)
    mask = (iota == idx).astype(jnp.float32)
    mask_ref[...] = mask

    psum_ref[0, ...] = jnp.sum(probs, axis=0, keepdims=True)
    usum_ref[0, ...] = jnp.sum(mask, axis=0, keepdims=True)


def _loss_kernel(psums_ref, usums_ref, loss_ref):
    rp = jnp.sum(psums_ref[...], axis=0) / TOKENS
    us = jnp.sum(usums_ref[...], axis=0) / TOKENS
    mm = jnp.max(rp)
    lse = jnp.log(jnp.sum(jnp.exp(rp - mm))) + mm
    logp = rp - lse
    loss_ref[...] = (-jnp.sum(us * logp, axis=1, keepdims=True)
                     * LOAD_BALANCING_LAMBDA)


def kernel(hidden_states, W):
    b, s, h = hidden_states.shape
    x = hidden_states.reshape(b * s, h)
    logits, mask, psums, usums = pl.pallas_call(
        _router_kernel,
        grid=(N_STEPS,),
        in_specs=[
            pl.BlockSpec((BLOCK_T, HIDDEN), lambda i: (i, 0)),
            pl.BlockSpec((NUM_EXPERTS, HIDDEN), lambda i: (0, 0)),
        ],
        out_specs=[
            pl.BlockSpec((BLOCK_T, NUM_EXPERTS), lambda i: (i, 0)),
            pl.BlockSpec((BLOCK_T, NUM_EXPERTS), lambda i: (i, 0)),
            pl.BlockSpec((1, 1, NUM_EXPERTS), lambda i: (i, 0, 0)),
            pl.BlockSpec((1, 1, NUM_EXPERTS), lambda i: (i, 0, 0)),
        ],
        out_shape=[
            jax.ShapeDtypeStruct((TOKENS, NUM_EXPERTS), jnp.float32),
            jax.ShapeDtypeStruct((TOKENS, NUM_EXPERTS), jnp.float32),
            jax.ShapeDtypeStruct((N_STEPS, 1, NUM_EXPERTS), jnp.float32),
            jax.ShapeDtypeStruct((N_STEPS, 1, NUM_EXPERTS), jnp.float32),
        ],
        compiler_params=pltpu.CompilerParams(
            dimension_semantics=("parallel",)),
    )(x, W)
    loss = pl.pallas_call(
        _loss_kernel,
        out_shape=jax.ShapeDtypeStruct((1, 1), jnp.float32),
    )(psums, usums)
    return (logits.reshape(b, s, NUM_EXPERTS),
            mask.reshape(b, s, NUM_EXPERTS),
            loss[0, 0])
